# baseline (speedup 1.0000x reference)
import functools

import jax
import jax.numpy as jnp
from jax import lax
from jax.experimental import pallas as pl
from jax.experimental.pallas import tpu as pltpu
from jax.experimental.pallas import tpu_sc as plsc

N_NODES = 10000
HID = 200
KF = 208
L = 16
NS = 16
CS = 64
G = 16
GC = G * CS
HALF = N_NODES // 2
ACC_R = 5120
RPT = ACC_R // NS
EPAD = 327680
CAP = 180224
BASE1 = EPAD - CAP


def _make_sc_spmm():
    ept = CAP // NS
    nsup = ept // GC

    mesh = plsc.VectorSubcoreMesh(core_axis_name="c", subcore_axis_name="s")

    @functools.partial(
        pl.kernel,
        mesh=mesh,
        compiler_params=pltpu.CompilerParams(use_tc_tiling_on_sc=False),
        out_type=jax.ShapeDtypeStruct((2 * ACC_R, KF), jnp.float32),
        scratch_types=[
            pltpu.VMEM((GC,), jnp.int32),
            pltpu.VMEM((G, CS), jnp.int32),
            pltpu.VMEM((GC,), jnp.float32),
            pltpu.VMEM((CS, KF), jnp.float32),
            pltpu.VMEM((CS, KF), jnp.float32),
            pltpu.VMEM((CS, KF), jnp.float32),
            pltpu.VMEM((CS, KF), jnp.float32),
            pltpu.VMEM_SHARED((ACC_R, KF), jnp.float32),
            pltpu.SemaphoreType.DMA,
            pltpu.SemaphoreType.DMA,
            pltpu.SemaphoreType.DMA,
            pltpu.SemaphoreType.DMA,
        ],
    )
    def sc_spmm(h_hbm, col_hbm, row2d_hbm, val_hbm, out_hbm,
                colv, rowv2, valv, buf0, buf1, buf2, buf3, acc,
                sem0, sem1, sem2, sem3):
        c = lax.axis_index("c")
        s = lax.axis_index("s")
        bufs = (buf0, buf1, buf2, buf3)
        sems = (sem0, sem1, sem2, sem3)

        def zrow(i, _):
            for j in range(KF // L):
                buf0[i, pl.ds(j * L, L)] = jnp.zeros((L,), jnp.float32)
            return 0
        lax.fori_loop(0, CS, zrow, 0)
        for k2 in range(RPT // CS):
            pltpu.sync_copy(buf0, acc.at[pl.ds(s * RPT + k2 * CS, CS)])
        plsc.subcore_barrier()

        def scale(buf, g):
            def grp(gg, _):
                vv = valv[pl.ds(g * CS + gg * L, L)]
                for i in range(L):
                    bv = jnp.full((L,), vv[i], jnp.float32)
                    e = gg * L + i
                    for j in range(KF // L):
                        buf[e, pl.ds(j * L, L)] = buf[e, pl.ds(j * L, L)] * bv
                return 0
            lax.fori_loop(0, CS // L, grp, 0)

        def gather(g, k):
            return pltpu.async_copy(
                h_hbm.at[colv.at[pl.ds(g * CS, CS)]], bufs[k], sems[k])

        def process(g, k, prefetch):
            pltpu.make_async_copy(
                h_hbm.at[colv.at[pl.ds(g * CS, CS)]], bufs[k], sems[k]).wait()
            if prefetch:
                gather(g + 3, (k + 3) % 4)
            scale(bufs[k], g)
            pltpu.sync_copy(bufs[k], acc.at[rowv2.at[g]], add=True)

        def super_body(si, _):
            sbase = c * CAP + s * ept + si * GC
            pltpu.sync_copy(col_hbm.at[pl.ds(sbase, GC)], colv)
            pltpu.sync_copy(row2d_hbm.at[pl.ds(sbase // CS, G)], rowv2)
            pltpu.sync_copy(val_hbm.at[pl.ds(sbase, GC)], valv)
            for k in range(3):
                gather(k, k)

            def quad(gq, _):
                for k in range(4):
                    process(4 * gq + k, k, True)
                return 0
            lax.fori_loop(0, G // 4 - 1, quad, 0)
            process(G - 4, 0, True)
            process(G - 3, 1, False)
            process(G - 2, 2, False)
            process(G - 1, 3, False)
            return 0
        lax.fori_loop(0, nsup, super_body, 0)

        plsc.subcore_barrier()
        for k2 in range(RPT // CS):
            r0 = s * RPT + k2 * CS
            pltpu.sync_copy(acc.at[pl.ds(r0, CS)],
                            out_hbm.at[pl.ds(c * ACC_R + r0, CS)])

    return sc_spmm


def _padcols(u, off):
    bm = u.shape[0]
    return jnp.concatenate(
        [u[:, off:off + HID], jnp.zeros((bm, KF - HID), jnp.float32)], axis=1)


def _mm1_body(x_ref, w_ref, b_ref, h0_ref, h1_ref, h2_ref):
    u = jnp.dot(x_ref[...], w_ref[...], preferred_element_type=jnp.float32)
    u = jnp.maximum(u + b_ref[...], 0.0)
    h0_ref[...] = u[:, 0:HID]
    h1_ref[...] = _padcols(u, HID)
    h2_ref[...] = _padcols(u, 2 * HID)


def _mm1(x, w, b, bm=1000):
    m, k = x.shape
    n = w.shape[1]
    return pl.pallas_call(
        _mm1_body,
        grid=(m // bm,),
        in_specs=[
            pl.BlockSpec((bm, k), lambda i: (i, 0)),
            pl.BlockSpec((k, n), lambda i: (0, 0)),
            pl.BlockSpec((1, n), lambda i: (0, 0)),
        ],
        out_specs=[
            pl.BlockSpec((bm, HID), lambda i: (i, 0)),
            pl.BlockSpec((bm, KF), lambda i: (i, 0)),
            pl.BlockSpec((bm, KF), lambda i: (i, 0)),
        ],
        out_shape=[
            jax.ShapeDtypeStruct((m, HID), jnp.float32),
            jax.ShapeDtypeStruct((m, KF), jnp.float32),
            jax.ShapeDtypeStruct((m, KF), jnp.float32),
        ],
    )(x, w, b)


def _mm2_body(h0_ref, s1_ref, s3_ref, w0_ref, w1_ref, w2_ref,
              b0_ref, p1_ref, p2_ref):
    acc = jnp.dot(h0_ref[...], w0_ref[...], preferred_element_type=jnp.float32)
    acc += jnp.dot(s1_ref[...], w1_ref[...], preferred_element_type=jnp.float32)
    acc += jnp.dot(s3_ref[...], w2_ref[...], preferred_element_type=jnp.float32)
    b0_ref[...] = acc[:, 0:HID]
    p1_ref[...] = _padcols(acc, HID)
    p2_ref[...] = _padcols(acc, 2 * HID)


def _mm2(h0, s1, s3, w0, w1, w2, bm=1000):
    m = h0.shape[0]
    n = w0.shape[1]
    return pl.pallas_call(
        _mm2_body,
        grid=(m // bm,),
        in_specs=[
            pl.BlockSpec((bm, HID), lambda i: (i, 0)),
            pl.BlockSpec((bm, KF), lambda i: (i, 0)),
            pl.BlockSpec((bm, KF), lambda i: (i, 0)),
            pl.BlockSpec((HID, n), lambda i: (0, 0)),
            pl.BlockSpec((KF, n), lambda i: (0, 0)),
            pl.BlockSpec((KF, n), lambda i: (0, 0)),
        ],
        out_specs=[
            pl.BlockSpec((bm, HID), lambda i: (i, 0)),
            pl.BlockSpec((bm, KF), lambda i: (i, 0)),
            pl.BlockSpec((bm, KF), lambda i: (i, 0)),
        ],
        out_shape=[
            jax.ShapeDtypeStruct((m, HID), jnp.float32),
            jax.ShapeDtypeStruct((m, KF), jnp.float32),
            jax.ShapeDtypeStruct((m, KF), jnp.float32),
        ],
    )(h0, s1, s3, w0, w1, w2)


def _final_body(b0_ref, t1_ref, t3_ref, w0_ref, w1_ref, w2_ref,
                bb0_ref, bb1_ref, bb2_ref, fcb_ref, ne_ref, pr_ref):
    ne = jnp.dot(b0_ref[...], w0_ref[...], preferred_element_type=jnp.float32)
    ne += jnp.dot(t1_ref[...], w1_ref[...], preferred_element_type=jnp.float32)
    ne += jnp.dot(t3_ref[...], w2_ref[...], preferred_element_type=jnp.float32)
    bias = fcb_ref[...]
    bias += jnp.dot(bb0_ref[...], w0_ref[...], preferred_element_type=jnp.float32)
    bias += jnp.dot(bb1_ref[...], w1_ref[...], preferred_element_type=jnp.float32)
    bias += jnp.dot(bb2_ref[...], w2_ref[...], preferred_element_type=jnp.float32)
    ne = ne + bias
    mx = jnp.max(ne, axis=1, keepdims=True)
    lse = jnp.log(jnp.sum(jnp.exp(ne - mx), axis=1, keepdims=True)) + mx
    ne_ref[...] = ne
    pr_ref[...] = ne - lse


def _final(b0, t1, t3, w0, w1, w2, bb0, bb1, bb2, fcb, bm=1000):
    m = b0.shape[0]
    n = w0.shape[1]
    return pl.pallas_call(
        _final_body,
        grid=(m // bm,),
        in_specs=[
            pl.BlockSpec((bm, HID), lambda i: (i, 0)),
            pl.BlockSpec((bm, KF), lambda i: (i, 0)),
            pl.BlockSpec((bm, KF), lambda i: (i, 0)),
            pl.BlockSpec((HID, n), lambda i: (0, 0)),
            pl.BlockSpec((KF, n), lambda i: (0, 0)),
            pl.BlockSpec((KF, n), lambda i: (0, 0)),
            pl.BlockSpec((1, HID), lambda i: (0, 0)),
            pl.BlockSpec((1, KF), lambda i: (0, 0)),
            pl.BlockSpec((1, KF), lambda i: (0, 0)),
            pl.BlockSpec((1, n), lambda i: (0, 0)),
        ],
        out_specs=[
            pl.BlockSpec((bm, n), lambda i: (i, 0)),
            pl.BlockSpec((bm, n), lambda i: (i, 0)),
        ],
        out_shape=[
            jax.ShapeDtypeStruct((m, n), jnp.float32),
            jax.ShapeDtypeStruct((m, n), jnp.float32),
        ],
    )(b0, t1, t3, w0, w1, w2, bb0, bb1, bb2, fcb)


def _pad_rows(w):
    return jnp.pad(w, ((0, KF - HID), (0, 0)))


def _unpad(o):
    return jnp.concatenate([o[0:HALF], o[ACC_R:ACC_R + HALF]])


def kernel(adj_index, adj_values, features, W_up0, b_up0, W_up1, b_up1,
           W_up2, b_up2, W_bot0, b_bot0, W_bot1, b_bot1, W_bot2, b_bot2,
           fc_w, fc_b):
    e = adj_values.shape[0]
    row = adj_index[0].astype(jnp.int32)
    col = adj_index[1].astype(jnp.int32)
    val = adj_values

    b = row < HALF
    bi = b.astype(jnp.int32)
    ct = jnp.cumsum(bi)
    cf = jnp.cumsum(1 - bi)
    nt = ct[-1]
    dest = jnp.where(b, ct - 1, nt + cf - 1)
    zi = jnp.zeros((e,), jnp.int32)
    zf = jnp.zeros((e,), jnp.float32)
    row_p = zi.at[dest].set(row)
    col_p = zi.at[dest].set(col)
    val_p = zf.at[dest].set(val)

    pad = EPAD - e
    row_p = jnp.pad(row_p, (0, pad))
    col_p = jnp.pad(col_p, (0, pad))
    val_p = jnp.pad(val_p, (0, pad))

    in0 = row_p < HALF
    row0 = jnp.where(in0, row_p, 0)
    val0 = jnp.where(in0, val_p, 0.0)
    row1 = jnp.where(in0, 0, row_p - HALF)
    val1 = jnp.where(in0, 0.0, val_p)
    row_sc = jnp.concatenate([row0[0:CAP], row1[BASE1:EPAD]]).reshape(-1, CS)
    val_sc = jnp.concatenate([val0[0:CAP], val1[BASE1:EPAD]])
    col_a = jnp.concatenate([col_p[0:CAP], col_p[BASE1:EPAD]])
    col_pm = jnp.where(col_p < HALF, col_p, col_p + (ACC_R - HALF))
    col_b = jnp.concatenate([col_pm[0:CAP], col_pm[BASE1:EPAD]])

    spmm = _make_sc_spmm()

    Wup = jnp.concatenate([W_up0, W_up1, W_up2], axis=1)
    bup = jnp.concatenate([b_up0, b_up1, b_up2], axis=1)
    h0, h1p, h2p = _mm1(features, Wup, bup)

    s1 = spmm(h1p, col_a, row_sc, val_sc)
    s2 = spmm(h2p, col_a, row_sc, val_sc)
    s3 = spmm(s2, col_b, row_sc, val_sc)

    Wbot = jnp.concatenate([W_bot0, W_bot1, W_bot2], axis=1)
    b0, p1p, p2p = _mm2(h0, _unpad(s1), _unpad(s3), Wbot[0:HID],
                        _pad_rows(Wbot[HID:2 * HID]),
                        _pad_rows(Wbot[2 * HID:3 * HID]))

    t1 = spmm(p1p, col_a, row_sc, val_sc)
    t2 = spmm(p2p, col_a, row_sc, val_sc)
    t3 = spmm(t2, col_b, row_sc, val_sc)

    bb1 = jnp.pad(b_bot1, ((0, 0), (0, KF - HID)))
    bb2 = jnp.pad(b_bot2, ((0, 0), (0, KF - HID)))
    node_emb, predictions = _final(
        b0, _unpad(t1), _unpad(t3), fc_w[0:HID],
        _pad_rows(fc_w[HID:2 * HID]), _pad_rows(fc_w[2 * HID:3 * HID]),
        b_bot0, bb1, bb2, fc_b[None, :])
    return (node_emb, predictions)


# speedup vs baseline: 1.9870x; 1.9870x over previous
import functools

import jax
import jax.numpy as jnp
from jax import lax
from jax.experimental import pallas as pl
from jax.experimental.pallas import tpu as pltpu
from jax.experimental.pallas import tpu_sc as plsc

N_NODES = 10000
HID = 200
KH = 112
L = 16
NS = 16
C = 128
ROWS_PER_TILE = N_NODES // NS
WB = 125


G = 16
GC = G * C
CS = 64


def _make_sc_spmm(epad):
    ept = epad // NS
    nch = ept // C
    nsup = nch // G
    assert ept % (G * C) == 0

    mesh = plsc.VectorSubcoreMesh(core_axis_name="c", subcore_axis_name="s")

    @functools.partial(
        pl.kernel,
        mesh=mesh,
        compiler_params=pltpu.CompilerParams(use_tc_tiling_on_sc=False),
        out_type=jax.ShapeDtypeStruct((2 * N_NODES, KH), jnp.float32),
        scratch_types=[
            pltpu.VMEM((GC,), jnp.int32),
            pltpu.VMEM((GC // CS, CS), jnp.int32),
            pltpu.VMEM((GC,), jnp.float32),
            pltpu.VMEM((CS, KH), jnp.float32),
            pltpu.VMEM((CS, KH), jnp.float32),
            pltpu.VMEM((CS, KH), jnp.float32),
            pltpu.VMEM((CS, KH), jnp.float32),
            pltpu.VMEM_SHARED((N_NODES, KH), jnp.float32),
            pltpu.SemaphoreType.DMA,
            pltpu.SemaphoreType.DMA,
            pltpu.SemaphoreType.DMA,
            pltpu.SemaphoreType.DMA,
        ],
    )
    def sc_spmm(h_hbm, col2_hbm, row2d_hbm, val_hbm, out_hbm,
                colv, rowv2, valv, buf0, buf1, buf2, buf3, acc,
                sem0, sem1, sem2, sem3):
        c = lax.axis_index("c")
        s = lax.axis_index("s")

        def zrow(i, _):
            for j in range(KH // L):
                buf0[i, pl.ds(j * L, L)] = jnp.zeros((L,), jnp.float32)
            return 0
        lax.fori_loop(0, C, zrow, 0)
        for k2 in range(ROWS_PER_TILE // WB):
            pltpu.sync_copy(buf0.at[pl.ds(0, WB)],
                            acc.at[pl.ds(s * ROWS_PER_TILE + k2 * WB, WB)])
        plsc.subcore_barrier()

        bufs = (buf0, buf1, buf2, buf3)
        sems = (sem0, sem1, sem2, sem3)
        ncs = GC // CS

        def scale(buf, g):
            def grp(gg, _):
                vv = valv[pl.ds(g * CS + gg * L, L)]
                for i in range(L):
                    bv = jnp.full((L,), vv[i], jnp.float32)
                    e = gg * L + i
                    for j in range(KH // L):
                        buf[e, pl.ds(j * L, L)] = buf[e, pl.ds(j * L, L)] * bv
                return 0
            lax.fori_loop(0, CS // L, grp, 0)

        def gather(g, k):
            return pltpu.async_copy(
                h_hbm.at[colv.at[pl.ds(g * CS, CS)]], bufs[k], sems[k])

        def process(g, k, prefetch):
            pltpu.make_async_copy(
                h_hbm.at[colv.at[pl.ds(g * CS, CS)]], bufs[k], sems[k]).wait()
            if prefetch:
                gather(g + 3, (k + 3) % 4)
            scale(bufs[k], g)
            pltpu.sync_copy(bufs[k], acc.at[rowv2.at[g]], add=True)

        def super_body(si, _):
            sbase = s * ept + si * GC
            pltpu.sync_copy(col2_hbm.at[pl.ds(c * epad + sbase, GC)], colv)
            pltpu.sync_copy(row2d_hbm.at[pl.ds(s * (ept // CS) + si * ncs,
                                               ncs)], rowv2)
            pltpu.sync_copy(val_hbm.at[pl.ds(sbase, GC)], valv)
            for k in range(3):
                gather(k, k)

            def quad(gq, _):
                for k in range(4):
                    process(4 * gq + k, k, True)
                return 0
            lax.fori_loop(0, ncs // 4 - 1, quad, 0)
            process(ncs - 4, 0, True)
            process(ncs - 3, 1, False)
            process(ncs - 2, 2, False)
            process(ncs - 1, 3, False)
            return 0
        lax.fori_loop(0, nsup, super_body, 0)

        plsc.subcore_barrier()
        for k2 in range(ROWS_PER_TILE // WB):
            r0 = s * ROWS_PER_TILE + k2 * WB
            pltpu.sync_copy(acc.at[pl.ds(r0, WB)],
                            out_hbm.at[pl.ds(c * N_NODES + r0, WB)])

    return sc_spmm


def _split_stack(u, off):
    bm = u.shape[0]
    lo = u[:, off:off + KH]
    hi = jnp.concatenate(
        [u[:, off + KH:off + HID],
         jnp.zeros((bm, 2 * KH - HID), jnp.float32)], axis=1)
    return jnp.stack([lo, hi])


def _mm1_body(x_ref, w_ref, b_ref, h0_ref, h1_ref, h2_ref):
    u = jnp.dot(x_ref[...], w_ref[...], preferred_element_type=jnp.float32)
    u = jnp.maximum(u + b_ref[...], 0.0)
    h0_ref[...] = u[:, 0:HID]
    h1_ref[...] = _split_stack(u, HID)
    h2_ref[...] = _split_stack(u, 2 * HID)


def _mm1(x, w, b, bm=1000):
    m, k = x.shape
    n = w.shape[1]
    return pl.pallas_call(
        _mm1_body,
        grid=(m // bm,),
        in_specs=[
            pl.BlockSpec((bm, k), lambda i: (i, 0)),
            pl.BlockSpec((k, n), lambda i: (0, 0)),
            pl.BlockSpec((1, n), lambda i: (0, 0)),
        ],
        out_specs=[
            pl.BlockSpec((bm, HID), lambda i: (i, 0)),
            pl.BlockSpec((2, bm, KH), lambda i: (0, i, 0)),
            pl.BlockSpec((2, bm, KH), lambda i: (0, i, 0)),
        ],
        out_shape=[
            jax.ShapeDtypeStruct((m, HID), jnp.float32),
            jax.ShapeDtypeStruct((2, m, KH), jnp.float32),
            jax.ShapeDtypeStruct((2, m, KH), jnp.float32),
        ],
    )(x, w, b)


def _mm2_body(h0_ref, s1l_ref, s1h_ref, s3l_ref, s3h_ref,
              w0_ref, w1l_ref, w1h_ref, w2l_ref, w2h_ref,
              b0_ref, p1_ref, p2_ref):
    acc = jnp.dot(h0_ref[...], w0_ref[...], preferred_element_type=jnp.float32)
    acc += jnp.dot(s1l_ref[...], w1l_ref[...], preferred_element_type=jnp.float32)
    acc += jnp.dot(s1h_ref[...], w1h_ref[...], preferred_element_type=jnp.float32)
    acc += jnp.dot(s3l_ref[...], w2l_ref[...], preferred_element_type=jnp.float32)
    acc += jnp.dot(s3h_ref[...], w2h_ref[...], preferred_element_type=jnp.float32)
    b0_ref[...] = acc[:, 0:HID]
    p1_ref[...] = _split_stack(acc, HID)
    p2_ref[...] = _split_stack(acc, 2 * HID)


def _mm2(h0, s1, s3, w0, w1l, w1h, w2l, w2h, bm=1000):
    m = h0.shape[0]
    n = w0.shape[1]
    nb = m // bm
    return pl.pallas_call(
        _mm2_body,
        grid=(nb,),
        in_specs=[
            pl.BlockSpec((bm, HID), lambda i: (i, 0)),
            pl.BlockSpec((bm, KH), lambda i: (i, 0)),
            pl.BlockSpec((bm, KH), lambda i, nb=nb: (nb + i, 0)),
            pl.BlockSpec((bm, KH), lambda i: (i, 0)),
            pl.BlockSpec((bm, KH), lambda i, nb=nb: (nb + i, 0)),
            pl.BlockSpec((HID, n), lambda i: (0, 0)),
            pl.BlockSpec((KH, n), lambda i: (0, 0)),
            pl.BlockSpec((KH, n), lambda i: (0, 0)),
            pl.BlockSpec((KH, n), lambda i: (0, 0)),
            pl.BlockSpec((KH, n), lambda i: (0, 0)),
        ],
        out_specs=[
            pl.BlockSpec((bm, HID), lambda i: (i, 0)),
            pl.BlockSpec((2, bm, KH), lambda i: (0, i, 0)),
            pl.BlockSpec((2, bm, KH), lambda i: (0, i, 0)),
        ],
        out_shape=[
            jax.ShapeDtypeStruct((m, HID), jnp.float32),
            jax.ShapeDtypeStruct((2, m, KH), jnp.float32),
            jax.ShapeDtypeStruct((2, m, KH), jnp.float32),
        ],
    )(h0, s1, s1, s3, s3, w0, w1l, w1h, w2l, w2h)


def _final_body(b0_ref, t1l_ref, t1h_ref, t3l_ref, t3h_ref,
                w0_ref, w1l_ref, w1h_ref, w2l_ref, w2h_ref,
                bb0_ref, bb1l_ref, bb1h_ref, bb2l_ref, bb2h_ref, fcb_ref,
                ne_ref, pr_ref):
    ne = jnp.dot(b0_ref[...], w0_ref[...], preferred_element_type=jnp.float32)
    ne += jnp.dot(t1l_ref[...], w1l_ref[...], preferred_element_type=jnp.float32)
    ne += jnp.dot(t1h_ref[...], w1h_ref[...], preferred_element_type=jnp.float32)
    ne += jnp.dot(t3l_ref[...], w2l_ref[...], preferred_element_type=jnp.float32)
    ne += jnp.dot(t3h_ref[...], w2h_ref[...], preferred_element_type=jnp.float32)
    bias = fcb_ref[...]
    bias += jnp.dot(bb0_ref[...], w0_ref[...], preferred_element_type=jnp.float32)
    bias += jnp.dot(bb1l_ref[...], w1l_ref[...], preferred_element_type=jnp.float32)
    bias += jnp.dot(bb1h_ref[...], w1h_ref[...], preferred_element_type=jnp.float32)
    bias += jnp.dot(bb2l_ref[...], w2l_ref[...], preferred_element_type=jnp.float32)
    bias += jnp.dot(bb2h_ref[...], w2h_ref[...], preferred_element_type=jnp.float32)
    ne = ne + bias
    mx = jnp.max(ne, axis=1, keepdims=True)
    lse = jnp.log(jnp.sum(jnp.exp(ne - mx), axis=1, keepdims=True)) + mx
    ne_ref[...] = ne
    pr_ref[...] = ne - lse


def _final(b0, t1, t3, w0, w1l, w1h, w2l, w2h,
           bb0, bb1l, bb1h, bb2l, bb2h, fcb, bm=1000):
    m = b0.shape[0]
    n = w0.shape[1]
    nb = m // bm
    return pl.pallas_call(
        _final_body,
        grid=(nb,),
        in_specs=[
            pl.BlockSpec((bm, HID), lambda i: (i, 0)),
            pl.BlockSpec((bm, KH), lambda i: (i, 0)),
            pl.BlockSpec((bm, KH), lambda i, nb=nb: (nb + i, 0)),
            pl.BlockSpec((bm, KH), lambda i: (i, 0)),
            pl.BlockSpec((bm, KH), lambda i, nb=nb: (nb + i, 0)),
            pl.BlockSpec((HID, n), lambda i: (0, 0)),
            pl.BlockSpec((KH, n), lambda i: (0, 0)),
            pl.BlockSpec((KH, n), lambda i: (0, 0)),
            pl.BlockSpec((KH, n), lambda i: (0, 0)),
            pl.BlockSpec((KH, n), lambda i: (0, 0)),
            pl.BlockSpec((1, HID), lambda i: (0, 0)),
            pl.BlockSpec((1, KH), lambda i: (0, 0)),
            pl.BlockSpec((1, KH), lambda i: (0, 0)),
            pl.BlockSpec((1, KH), lambda i: (0, 0)),
            pl.BlockSpec((1, KH), lambda i: (0, 0)),
            pl.BlockSpec((1, n), lambda i: (0, 0)),
        ],
        out_specs=[
            pl.BlockSpec((bm, n), lambda i: (i, 0)),
            pl.BlockSpec((bm, n), lambda i: (i, 0)),
        ],
        out_shape=[
            jax.ShapeDtypeStruct((m, n), jnp.float32),
            jax.ShapeDtypeStruct((m, n), jnp.float32),
        ],
    )(b0, t1, t1, t3, t3, w0, w1l, w1h, w2l, w2h,
      bb0, bb1l, bb1h, bb2l, bb2h, fcb)


def _wsplit(w):
    lo = w[0:KH]
    hi = jnp.pad(w[KH:HID], ((0, 2 * KH - HID), (0, 0)))
    return lo, hi


def _bsplit(b):
    lo = b[:, 0:KH]
    hi = jnp.pad(b[:, KH:HID], ((0, 0), (0, 2 * KH - HID)))
    return lo, hi


def kernel(adj_index, adj_values, features, W_up0, b_up0, W_up1, b_up1,
           W_up2, b_up2, W_bot0, b_bot0, W_bot1, b_bot1, W_bot2, b_bot2,
           fc_w, fc_b):
    n = N_NODES
    e = adj_values.shape[0]
    epad = ((e + NS * GC - 1) // (NS * GC)) * (NS * GC)
    pad = epad - e
    row2d = jnp.pad(adj_index[0].astype(jnp.int32), (0, pad)).reshape(-1, CS)
    colp = jnp.pad(adj_index[1].astype(jnp.int32), (0, pad))
    col2 = jnp.concatenate([colp, colp + n])
    val = jnp.pad(adj_values, (0, pad))

    spmm = _make_sc_spmm(epad)

    Wup = jnp.concatenate([W_up0, W_up1, W_up2], axis=1)
    bup = jnp.concatenate([b_up0, b_up1, b_up2], axis=1)
    h0, h1s, h2s = _mm1(features, Wup, bup)

    s1 = spmm(h1s.reshape(2 * n, KH), col2, row2d, val)
    s2 = spmm(h2s.reshape(2 * n, KH), col2, row2d, val)
    s3 = spmm(s2, col2, row2d, val)

    Wbot = jnp.concatenate([W_bot0, W_bot1, W_bot2], axis=1)
    w1l, w1h = _wsplit(Wbot[HID:2 * HID])
    w2l, w2h = _wsplit(Wbot[2 * HID:3 * HID])
    b0, p1s, p2s = _mm2(h0, s1, s3, Wbot[0:HID], w1l, w1h, w2l, w2h)

    t1 = spmm(p1s.reshape(2 * n, KH), col2, row2d, val)
    t2 = spmm(p2s.reshape(2 * n, KH), col2, row2d, val)
    t3 = spmm(t2, col2, row2d, val)

    f1l, f1h = _wsplit(fc_w[HID:2 * HID])
    f2l, f2h = _wsplit(fc_w[2 * HID:3 * HID])
    bb1l, bb1h = _bsplit(b_bot1)
    bb2l, bb2h = _bsplit(b_bot2)
    node_emb, predictions = _final(
        b0, t1, t3, fc_w[0:HID], f1l, f1h, f2l, f2h,
        b_bot0, bb1l, bb1h, bb2l, bb2h, fc_b[None, :])
    return (node_emb, predictions)
